# baseline (speedup 1.0000x reference)
import functools
import math

import jax
import jax.numpy as jnp
from jax import lax
from jax.experimental import pallas as pl
from jax.experimental.pallas import tpu as pltpu
from jax.experimental.pallas import tpu_sc as plsc

N = 5000
NPAD = 5120
NCLS = 81
NC = 80
CAP = 112
DETS = 100
SCORE_THRESH = 0.05
NMS_THRESH = 0.5
IMG_W = 1024.0
IMG_H = 1024.0
BBOX_XFORM_CLIP = float(math.log(1000.0 / 16.0))
NEG = -1e30
L = 16
BLK = 1024


def _dense_body(lg_ref, dx_ref, dy_ref, dw_ref, dh_ref, pr_ref,
                s_ref, x1_ref, y1_ref, x2_ref, y2_ref):
    l = lg_ref[...]
    m = jnp.max(l, axis=0, keepdims=True)
    e = jnp.exp(l - m)
    den = jnp.sum(e, axis=0, keepdims=True)
    s_ref[...] = e[1:, :] / den

    px1 = pr_ref[0:1, :]
    py1 = pr_ref[1:2, :]
    px2 = pr_ref[2:3, :]
    py2 = pr_ref[3:4, :]
    w = px2 - px1 + 1.0
    h = py2 - py1 + 1.0
    cx = px1 + 0.5 * w
    cy = py1 + 0.5 * h
    dx = dx_ref[...] / 10.0
    dy = dy_ref[...] / 10.0
    dw = jnp.minimum(dw_ref[...] / 5.0, BBOX_XFORM_CLIP)
    dh = jnp.minimum(dh_ref[...] / 5.0, BBOX_XFORM_CLIP)
    pcx = dx * w + cx
    pcy = dy * h + cy
    pw = jnp.exp(dw) * w
    ph = jnp.exp(dh) * h
    x1_ref[...] = jnp.clip(pcx - 0.5 * pw, 0.0, IMG_W - 1.0)
    y1_ref[...] = jnp.clip(pcy - 0.5 * ph, 0.0, IMG_H - 1.0)
    x2_ref[...] = jnp.clip(pcx + 0.5 * pw - 1.0, 0.0, IMG_W - 1.0)
    y2_ref[...] = jnp.clip(pcy + 0.5 * ph - 1.0, 0.0, IMG_H - 1.0)


def _dense(lg, dx, dy, dw, dh, prop, interpret=False):
    nblk = NPAD // BLK
    cm81 = pl.BlockSpec((NCLS, BLK), lambda j: (0, j))
    cm = pl.BlockSpec((NC, BLK), lambda j: (0, j))
    return pl.pallas_call(
        _dense_body,
        grid=(nblk,),
        in_specs=[
            cm81,
            cm, cm, cm, cm,
            pl.BlockSpec((8, BLK), lambda j: (0, j)),
        ],
        out_specs=[cm] * 5,
        out_shape=[jax.ShapeDtypeStruct((NC, NPAD), jnp.float32)] * 5,
        interpret=interpret,
    )(lg, dx, dy, dw, dh, prop)


UNR = 4


def _merge_amax(pairs):
    rm, ri = pairs[0]
    for rm2, ri2 in pairs[1:]:
        upd = rm2 > rm
        rm = jnp.where(upd, rm2, rm)
        ri = jnp.where(upd, ri2, ri)
    return rm, ri


def _nms_body(s_hbm, x1_hbm, y1_hbm, x2_hbm, y2_hbm,
              ks_hbm, kx1_hbm, ky1_hbm, kx2_hbm, ky2_hbm,
              in_s, in_x1, in_y1, in_x2, in_y2,
              c_s, c_x1, c_y1, c_x2, c_y2, sbuf,
              ko_scr, ko_x1, ko_y1, ko_x2, ko_y2, sem, cnt_smem):
    lane = lax.iota(jnp.int32, L)
    rm0 = jnp.full((L,), NEG, jnp.float32)
    ri0 = jnp.zeros((L,), jnp.int32)

    def process(cidx):
        cps = [pltpu.async_copy(s_hbm.at[cidx], in_s, sem),
               pltpu.async_copy(x1_hbm.at[cidx], in_x1, sem),
               pltpu.async_copy(y1_hbm.at[cidx], in_y1, sem),
               pltpu.async_copy(x2_hbm.at[cidx], in_x2, sem),
               pltpu.async_copy(y2_hbm.at[cidx], in_y2, sem)]
        for cp in cps:
            cp.wait()

        def comp(i, cnt_v):
            sl = pl.ds(i * L, L)
            v = in_s[sl]
            m = v > SCORE_THRESH
            cs = plsc.cumsum(jnp.where(m, jnp.int32(1), jnp.int32(0)))
            idx = cnt_v + cs - 1
            plsc.store_scatter(c_s, [idx], v, mask=m)
            plsc.store_scatter(c_x1, [idx], in_x1[sl], mask=m)
            plsc.store_scatter(c_y1, [idx], in_y1[sl], mask=m)
            plsc.store_scatter(c_x2, [idx], in_x2[sl], mask=m)
            plsc.store_scatter(c_y2, [idx], in_y2[sl], mask=m)
            return cnt_v + plsc.all_reduce_population_count(m)

        cnt_v = plsc.parallel_loop(
            0, NPAD // L, unroll=UNR,
            carry=jnp.zeros((L,), jnp.int32))(comp)
        v_cnt = cnt_v[0]
        nv = (v_cnt + L - 1) // L

        pidx = v_cnt + lane
        pm = pidx < nv * L
        plsc.store_scatter(c_s, [jnp.minimum(pidx, NPAD - 1)],
                           jnp.full((L,), NEG, jnp.float32), mask=pm)

        for j in range(CAP // L):
            ko_scr[pl.ds(j * L, L)] = jnp.full((L,), NEG, jnp.float32)

        def vextract(ri, rm, ms_v):
            ffs = plsc.all_reduce_ffs(rm == ms_v)
            sbuf[pl.ds(0, L)] = ri
            return plsc.load_gather(sbuf, [ffs])

        def am(i, carry):
            rm, ri = carry
            v = c_s[pl.ds(i * L, L)]
            gi = i * L + lane
            upd = v > rm
            return jnp.where(upd, v, rm), jnp.where(upd, gi, ri)

        rm, ri = plsc.parallel_loop(0, nv, unroll=UNR,
                                    carry=(rm0, ri0))(am)
        ms_v = jnp.full((L,), jnp.max(rm))
        curv = vextract(ri, rm, ms_v)

        def w_body(kcnt, carry):
            curv_c, ms_v_c = carry
            bx1 = plsc.load_gather(c_x1, [curv_c])
            by1 = plsc.load_gather(c_y1, [curv_c])
            bx2 = plsc.load_gather(c_x2, [curv_c])
            by2 = plsc.load_gather(c_y2, [curv_c])
            bar = (bx2 - bx1 + 1.0) * (by2 - by1 + 1.0)

            kidx = jnp.full((L,), jnp.minimum(kcnt, CAP - 1))
            live = (lane == 0) & (ms_v_c > 0.0)
            plsc.store_scatter(ko_scr, [kidx], ms_v_c, mask=live)
            plsc.store_scatter(ko_x1, [kidx], bx1, mask=live)
            plsc.store_scatter(ko_y1, [kidx], by1, mask=live)
            plsc.store_scatter(ko_x2, [kidx], bx2, mask=live)
            plsc.store_scatter(ko_y2, [kidx], by2, mask=live)

            def sp(i, carry2):
                rm_i, ri_i = carry2
                sl = pl.ds(i * L, L)
                s = c_s[sl]
                x1c = c_x1[sl]
                y1c = c_y1[sl]
                x2c = c_x2[sl]
                y2c = c_y2[sl]
                ac = (x2c - x1c + 1.0) * (y2c - y1c + 1.0)
                xx1 = jnp.maximum(bx1, x1c)
                yy1 = jnp.maximum(by1, y1c)
                xx2 = jnp.minimum(bx2, x2c)
                yy2 = jnp.minimum(by2, y2c)
                inter = (jnp.maximum(0.0, xx2 - xx1 + 1.0)
                         * jnp.maximum(0.0, yy2 - yy1 + 1.0))
                gi = i * L + lane
                supp = (3.0 * inter > bar + ac) | (gi == curv_c)
                ns = jnp.where(supp, NEG, s)
                c_s[sl] = ns
                upd = ns > rm_i
                return (jnp.where(upd, ns, rm_i), jnp.where(upd, gi, ri_i))

            rm2, ri2 = plsc.parallel_loop(0, nv, unroll=UNR,
                                          carry=(rm0, ri0))(sp)
            ms_v2 = jnp.full((L,), jnp.max(rm2))
            curv2 = vextract(ri2, rm2, ms_v2)
            return curv2, ms_v2

        lax.fori_loop(0, v_cnt, w_body, (curv, ms_v))

        ocps = [pltpu.async_copy(ko_scr, ks_hbm.at[cidx], sem),
                pltpu.async_copy(ko_x1, kx1_hbm.at[cidx], sem),
                pltpu.async_copy(ko_y1, ky1_hbm.at[cidx], sem),
                pltpu.async_copy(ko_x2, kx2_hbm.at[cidx], sem),
                pltpu.async_copy(ko_y2, ky2_hbm.at[cidx], sem)]
        for cp in ocps:
            cp.wait()

    sid = lax.axis_index("s")
    base = lax.axis_index("c") * (NC // 2)

    @pl.when(sid == 0)
    def _():
        cnt_smem[0] = jnp.int32(0)

    plsc.subcore_barrier()

    def st_cond(c):
        return c < NC // 2

    def st_body(c):
        process(base + c)
        return plsc.fetch_and_add(cnt_smem, jnp.int32(1), subcore_id=0)

    lax.while_loop(st_cond, st_body,
                   plsc.fetch_and_add(cnt_smem, jnp.int32(1), subcore_id=0))


def _nms(scm, x1cm, y1cm, x2cm, y2cm):
    mesh = plsc.VectorSubcoreMesh(core_axis_name="c", subcore_axis_name="s",
                                  num_cores=2, num_subcores=16)
    f32 = jnp.float32
    out = jax.ShapeDtypeStruct((NC, CAP), f32)
    return pl.kernel(
        _nms_body,
        out_type=[out] * 5,
        mesh=mesh,
        compiler_params=pltpu.CompilerParams(needs_layout_passes=False),
        scratch_types=(
            [pltpu.VMEM((NPAD,), f32)] * 5
            + [pltpu.VMEM((NPAD,), f32)] * 5
            + [pltpu.VMEM((L,), jnp.int32)]
            + [pltpu.VMEM((CAP,), f32)] * 5
            + [pltpu.SemaphoreType.DMA]
            + [pltpu.SMEM((1,), jnp.int32)]
        ),
    )(scm, x1cm, y1cm, x2cm, y2cm)


def _merge_body(ks, kx1, ky1, kx2, ky2, scm, x1cm, y1cm, x2cm, y2cm,
                ob_hbm, os_hbm, ol_hbm,
                ks_v, kx1_v, ky1_v, kx2_v, ky2_v,
                pos_v, tmp16, sbuf, ob_v, os_v, ol_v):
    wid = lax.axis_index("s") * 2 + lax.axis_index("c")
    lane = lax.iota(jnp.int32, L)
    zero16 = jnp.zeros((L,), jnp.int32)

    @pl.when(wid == 0)
    def _():
        pltpu.sync_copy(ks, ks_v)
        pltpu.sync_copy(kx1, kx1_v)
        pltpu.sync_copy(ky1, ky1_v)
        pltpu.sync_copy(kx2, kx2_v)
        pltpu.sync_copy(ky2, ky2_v)

        def first_elem(hbm):
            pltpu.sync_copy(hbm.at[0, pl.ds(0, L)], tmp16)
            return plsc.load_gather(tmp16, [zero16])

        d_scr = first_elem(scm)
        d_x1 = first_elem(x1cm)
        d_y1 = first_elem(y1cm)
        d_x2 = first_elem(x2cm)
        d_y2 = first_elem(y2cm)

        def init_head(j, _):
            cls16 = j * L + lane
            hs = plsc.load_gather(ks_v, [cls16, zero16])
            sbuf[pl.ds(j * L, L)] = hs
            return 0

        lax.fori_loop(0, NC // L, init_head, 0)
        heads0 = tuple(sbuf[pl.ds(j * L, L)] for j in range(NC // L))

        for j in range(CAP // L):
            pos_v[pl.ds(j * L, L)] = jnp.zeros((L,), jnp.int32)

        def step(t, carry):
            (h0, h1, h2, h3, h4), last = carry
            lcls, lscr, lx1, ly1, lx2, ly2 = last
            m16 = jnp.maximum(jnp.maximum(jnp.maximum(h0, h1),
                                          jnp.maximum(h2, h3)), h4)
            msv = jnp.full((L,), jnp.max(m16))
            f0 = plsc.all_reduce_ffs(h0 == msv)
            f1 = plsc.all_reduce_ffs(h1 == msv)
            f2 = plsc.all_reduce_ffs(h2 == msv)
            f3 = plsc.all_reduce_ffs(h3 == msv)
            f4 = plsc.all_reduce_ffs(h4 == msv)
            cstar = jnp.where(
                f0 < L, f0,
                jnp.where(f1 < L, L + f1,
                          jnp.where(f2 < L, 2 * L + f2,
                                    jnp.where(f3 < L, 3 * L + f3, 4 * L + f4))))
            ex = msv <= -1e29
            csv = jnp.where(ex, 0, cstar)
            pv = plsc.load_gather(pos_v, [csv])
            bx1 = plsc.load_gather(kx1_v, [csv, pv])
            by1 = plsc.load_gather(ky1_v, [csv, pv])
            bx2 = plsc.load_gather(kx2_v, [csv, pv])
            by2 = plsc.load_gather(ky2_v, [csv, pv])
            pn = jnp.minimum(pv + 1, CAP - 1)
            nh_v = plsc.load_gather(ks_v, [csv, pn])
            plsc.store_scatter(pos_v, [csv], pn, mask=lane == 0)

            sel_lane = lane == (csv - (csv // L) * L)
            cchunk = csv // L
            h0 = jnp.where(sel_lane & (cchunk == 0), nh_v, h0)
            h1 = jnp.where(sel_lane & (cchunk == 1), nh_v, h1)
            h2 = jnp.where(sel_lane & (cchunk == 2), nh_v, h2)
            h3 = jnp.where(sel_lane & (cchunk == 3), nh_v, h3)
            h4 = jnp.where(sel_lane & (cchunk == 4), nh_v, h4)

            ncls = jnp.where(ex, lcls, csv + 1)
            nscr = jnp.where(ex, lscr, msv)
            nx1 = jnp.where(ex, lx1, bx1)
            ny1 = jnp.where(ex, ly1, by1)
            nx2 = jnp.where(ex, lx2, bx2)
            ny2 = jnp.where(ex, ly2, by2)

            tv = jnp.full((L,), t)
            plsc.store_scatter(os_v, [tv], nscr, mask=lane == 0)
            plsc.store_scatter(ol_v, [tv], ncls, mask=lane == 0)
            bval = jnp.where(lane == 0, nx1,
                             jnp.where(lane == 1, ny1,
                                       jnp.where(lane == 2, nx2, ny2)))
            plsc.store_scatter(ob_v, [t * 4 + lane], bval, mask=lane < 4)
            return ((h0, h1, h2, h3, h4),
                    (ncls, nscr, nx1, ny1, nx2, ny2))

        ones_i = jnp.full((L,), 1, jnp.int32)
        lax.fori_loop(0, DETS, step,
                      (heads0,
                       (ones_i, d_scr, d_x1, d_y1, d_x2, d_y2)))

        pltpu.sync_copy(ob_v, ob_hbm)
        pltpu.sync_copy(os_v, os_hbm)
        pltpu.sync_copy(ol_v, ol_hbm)


def _merge(ks, kx1, ky1, kx2, ky2, scm, x1cm, y1cm, x2cm, y2cm):
    mesh = plsc.VectorSubcoreMesh(core_axis_name="c", subcore_axis_name="s",
                                  num_cores=2, num_subcores=16)
    f32 = jnp.float32
    i32 = jnp.int32
    return pl.kernel(
        _merge_body,
        compiler_params=pltpu.CompilerParams(needs_layout_passes=False),
        out_type=[
            jax.ShapeDtypeStruct((CAP * 4,), f32),
            jax.ShapeDtypeStruct((CAP,), f32),
            jax.ShapeDtypeStruct((CAP,), i32),
        ],
        mesh=mesh,
        scratch_types=(
            [pltpu.VMEM((NC, CAP), f32)] * 5
            + [pltpu.VMEM((128,), i32),
               pltpu.VMEM((L,), f32),
               pltpu.VMEM((NC,), f32),
               pltpu.VMEM((CAP * 4,), f32), pltpu.VMEM((CAP,), f32),
               pltpu.VMEM((CAP,), i32)]
        ),
    )(ks, kx1, ky1, kx2, ky2, scm, x1cm, y1cm, x2cm, y2cm)


_DIAG = 0


def kernel(class_logits, box_regression, proposals):
    pad = NPAD - N
    lg = jnp.pad(class_logits, ((0, pad), (0, 0))).T
    br = box_regression.reshape(N, NCLS, 4)
    dx = jnp.pad(br[:, 1:, 0], ((0, pad), (0, 0))).T
    dy = jnp.pad(br[:, 1:, 1], ((0, pad), (0, 0))).T
    dw = jnp.pad(br[:, 1:, 2], ((0, pad), (0, 0))).T
    dh = jnp.pad(br[:, 1:, 3], ((0, pad), (0, 0))).T
    prop = jnp.pad(proposals.T, ((0, 4), (0, pad)))

    scm, x1cm, y1cm, x2cm, y2cm = _dense(lg, dx, dy, dw, dh, prop)
    if _DIAG == 1:
        return (x1cm[0, :400].reshape(DETS, 4), scm[0, :DETS],
                y1cm[0, :DETS].astype(jnp.int32))
    ks, kx1, ky1, kx2, ky2 = _nms(scm, x1cm, y1cm, x2cm, y2cm)
    if _DIAG == 2:
        return (kx1[0, :400].reshape(DETS, 4), ks[0, :DETS],
                ky1[0, :DETS].astype(jnp.int32))
    ob, osc, olb = _merge(ks, kx1, ky1, kx2, ky2,
                          scm, x1cm, y1cm, x2cm, y2cm)
    det_boxes = ob[: DETS * 4].reshape(DETS, 4)
    return det_boxes, osc[:DETS], olb[:DETS]


# speedup vs baseline: 1.1604x; 1.1604x over previous
import functools
import math

import jax
import jax.numpy as jnp
from jax import lax
from jax.experimental import pallas as pl
from jax.experimental.pallas import tpu as pltpu
from jax.experimental.pallas import tpu_sc as plsc

N = 5000
NPAD = 5120
NCLS = 81
NC = 80
CAP = 112
DETS = 100
SCORE_THRESH = 0.05
NMS_THRESH = 0.5
IMG_W = 1024.0
IMG_H = 1024.0
BBOX_XFORM_CLIP = float(math.log(1000.0 / 16.0))
NEG = -1e30
L = 16
BLK = 1024


def _dense_body(lg_ref, br_ref, pr_ref,
                s_ref, x1_ref, y1_ref, x2_ref, y2_ref):
    l = lg_ref[...]
    m = jnp.max(l, axis=0, keepdims=True)
    e = jnp.exp(l - m)
    den = jnp.sum(e, axis=0, keepdims=True)
    s_ref[...] = e[1:, :] / den

    px1 = pr_ref[0:1, :]
    py1 = pr_ref[1:2, :]
    px2 = pr_ref[2:3, :]
    py2 = pr_ref[3:4, :]
    w = px2 - px1 + 1.0
    h = py2 - py1 + 1.0
    cx = px1 + 0.5 * w
    cy = py1 + 0.5 * h
    br = br_ref[...].reshape(NCLS, 4, -1)[1:]
    dx = br[:, 0, :] / 10.0
    dy = br[:, 1, :] / 10.0
    dw = jnp.minimum(br[:, 2, :] / 5.0, BBOX_XFORM_CLIP)
    dh = jnp.minimum(br[:, 3, :] / 5.0, BBOX_XFORM_CLIP)
    pcx = dx * w + cx
    pcy = dy * h + cy
    pw = jnp.exp(dw) * w
    ph = jnp.exp(dh) * h
    x1_ref[...] = jnp.clip(pcx - 0.5 * pw, 0.0, IMG_W - 1.0)
    y1_ref[...] = jnp.clip(pcy - 0.5 * ph, 0.0, IMG_H - 1.0)
    x2_ref[...] = jnp.clip(pcx + 0.5 * pw - 1.0, 0.0, IMG_W - 1.0)
    y2_ref[...] = jnp.clip(pcy + 0.5 * ph - 1.0, 0.0, IMG_H - 1.0)


def _dense(lg, brt, prop, interpret=False):
    nblk = NPAD // BLK
    cm81 = pl.BlockSpec((NCLS, BLK), lambda j: (0, j))
    cm = pl.BlockSpec((NC, BLK), lambda j: (0, j))
    return pl.pallas_call(
        _dense_body,
        grid=(nblk,),
        in_specs=[
            cm81,
            pl.BlockSpec((NCLS * 4, BLK), lambda j: (0, j)),
            pl.BlockSpec((8, BLK), lambda j: (0, j)),
        ],
        out_specs=[cm] * 5,
        out_shape=[jax.ShapeDtypeStruct((NC, NPAD), jnp.float32)] * 5,
        interpret=interpret,
    )(lg, brt, prop)


UNR = 4


def _merge_amax(pairs):
    rm, ri = pairs[0]
    for rm2, ri2 in pairs[1:]:
        upd = rm2 > rm
        rm = jnp.where(upd, rm2, rm)
        ri = jnp.where(upd, ri2, ri)
    return rm, ri


def _nms_body(s_hbm, x1_hbm, y1_hbm, x2_hbm, y2_hbm,
              ks_hbm, kx1_hbm, ky1_hbm, kx2_hbm, ky2_hbm,
              in_s, in_x1, in_y1, in_x2, in_y2,
              c_s, c_x1, c_y1, c_x2, c_y2, sbuf,
              ko_scr, ko_x1, ko_y1, ko_x2, ko_y2, sem, cnt_smem):
    lane = lax.iota(jnp.int32, L)
    rm0 = jnp.full((L,), NEG, jnp.float32)
    ri0 = jnp.zeros((L,), jnp.int32)

    def process(cidx):
        cps = [pltpu.async_copy(s_hbm.at[cidx], in_s, sem),
               pltpu.async_copy(x1_hbm.at[cidx], in_x1, sem),
               pltpu.async_copy(y1_hbm.at[cidx], in_y1, sem),
               pltpu.async_copy(x2_hbm.at[cidx], in_x2, sem),
               pltpu.async_copy(y2_hbm.at[cidx], in_y2, sem)]
        for cp in cps:
            cp.wait()

        def comp(i, cnt_v):
            sl = pl.ds(i * L, L)
            v = in_s[sl]
            m = v > SCORE_THRESH
            cs = plsc.cumsum(jnp.where(m, jnp.int32(1), jnp.int32(0)))
            idx = cnt_v + cs - 1
            plsc.store_scatter(c_s, [idx], v, mask=m)
            plsc.store_scatter(c_x1, [idx], in_x1[sl], mask=m)
            plsc.store_scatter(c_y1, [idx], in_y1[sl], mask=m)
            plsc.store_scatter(c_x2, [idx], in_x2[sl], mask=m)
            plsc.store_scatter(c_y2, [idx], in_y2[sl], mask=m)
            return cnt_v + plsc.all_reduce_population_count(m)

        cnt_v = plsc.parallel_loop(
            0, NPAD // L, unroll=UNR,
            carry=jnp.zeros((L,), jnp.int32))(comp)
        v_cnt = cnt_v[0]
        nv = (v_cnt + L - 1) // L

        pidx = v_cnt + lane
        pm = pidx < nv * L
        plsc.store_scatter(c_s, [jnp.minimum(pidx, NPAD - 1)],
                           jnp.full((L,), NEG, jnp.float32), mask=pm)

        for j in range(CAP // L):
            ko_scr[pl.ds(j * L, L)] = jnp.full((L,), NEG, jnp.float32)

        def vextract(ri, rm, ms_v):
            ffs = plsc.all_reduce_ffs(rm == ms_v)
            sbuf[pl.ds(0, L)] = ri
            return plsc.load_gather(sbuf, [ffs])

        def am(i, carry):
            rm, ri = carry
            v = c_s[pl.ds(i * L, L)]
            gi = i * L + lane
            upd = v > rm
            return jnp.where(upd, v, rm), jnp.where(upd, gi, ri)

        rm, ri = plsc.parallel_loop(0, nv, unroll=UNR,
                                    carry=(rm0, ri0))(am)
        ms_v = jnp.full((L,), jnp.max(rm))
        curv = vextract(ri, rm, ms_v)

        def w_body(kcnt, carry):
            curv_c, ms_v_c = carry
            bx1 = plsc.load_gather(c_x1, [curv_c])
            by1 = plsc.load_gather(c_y1, [curv_c])
            bx2 = plsc.load_gather(c_x2, [curv_c])
            by2 = plsc.load_gather(c_y2, [curv_c])
            bar = (bx2 - bx1 + 1.0) * (by2 - by1 + 1.0)

            kidx = jnp.full((L,), jnp.minimum(kcnt, CAP - 1))
            live = (lane == 0) & (ms_v_c > 0.0)
            plsc.store_scatter(ko_scr, [kidx], ms_v_c, mask=live)
            plsc.store_scatter(ko_x1, [kidx], bx1, mask=live)
            plsc.store_scatter(ko_y1, [kidx], by1, mask=live)
            plsc.store_scatter(ko_x2, [kidx], bx2, mask=live)
            plsc.store_scatter(ko_y2, [kidx], by2, mask=live)

            def sp(i, carry2):
                rm_i, ri_i = carry2
                sl = pl.ds(i * L, L)
                s = c_s[sl]
                x1c = c_x1[sl]
                y1c = c_y1[sl]
                x2c = c_x2[sl]
                y2c = c_y2[sl]
                ac = (x2c - x1c + 1.0) * (y2c - y1c + 1.0)
                xx1 = jnp.maximum(bx1, x1c)
                yy1 = jnp.maximum(by1, y1c)
                xx2 = jnp.minimum(bx2, x2c)
                yy2 = jnp.minimum(by2, y2c)
                inter = (jnp.maximum(0.0, xx2 - xx1 + 1.0)
                         * jnp.maximum(0.0, yy2 - yy1 + 1.0))
                gi = i * L + lane
                supp = (3.0 * inter > bar + ac) | (gi == curv_c)
                ns = jnp.where(supp, NEG, s)
                c_s[sl] = ns
                upd = ns > rm_i
                return (jnp.where(upd, ns, rm_i), jnp.where(upd, gi, ri_i))

            rm2, ri2 = plsc.parallel_loop(0, nv, unroll=UNR,
                                          carry=(rm0, ri0))(sp)
            ms_v2 = jnp.full((L,), jnp.max(rm2))
            curv2 = vextract(ri2, rm2, ms_v2)
            return curv2, ms_v2

        lax.fori_loop(0, v_cnt, w_body, (curv, ms_v))

        ocps = [pltpu.async_copy(ko_scr, ks_hbm.at[cidx], sem),
                pltpu.async_copy(ko_x1, kx1_hbm.at[cidx], sem),
                pltpu.async_copy(ko_y1, ky1_hbm.at[cidx], sem),
                pltpu.async_copy(ko_x2, kx2_hbm.at[cidx], sem),
                pltpu.async_copy(ko_y2, ky2_hbm.at[cidx], sem)]
        for cp in ocps:
            cp.wait()

    sid = lax.axis_index("s")
    base = lax.axis_index("c") * (NC // 2)

    @pl.when(sid == 0)
    def _():
        cnt_smem[0] = jnp.int32(0)

    plsc.subcore_barrier()

    def st_cond(c):
        return c < NC // 2

    def st_body(c):
        process(base + c)
        return plsc.fetch_and_add(cnt_smem, jnp.int32(1), subcore_id=0)

    lax.while_loop(st_cond, st_body,
                   plsc.fetch_and_add(cnt_smem, jnp.int32(1), subcore_id=0))


def _nms(scm, x1cm, y1cm, x2cm, y2cm):
    mesh = plsc.VectorSubcoreMesh(core_axis_name="c", subcore_axis_name="s",
                                  num_cores=2, num_subcores=16)
    f32 = jnp.float32
    out = jax.ShapeDtypeStruct((NC, CAP), f32)
    return pl.kernel(
        _nms_body,
        out_type=[out] * 5,
        mesh=mesh,
        compiler_params=pltpu.CompilerParams(needs_layout_passes=False),
        scratch_types=(
            [pltpu.VMEM((NPAD,), f32)] * 5
            + [pltpu.VMEM((NPAD,), f32)] * 5
            + [pltpu.VMEM((L,), jnp.int32)]
            + [pltpu.VMEM((CAP,), f32)] * 5
            + [pltpu.SemaphoreType.DMA]
            + [pltpu.SMEM((1,), jnp.int32)]
        ),
    )(scm, x1cm, y1cm, x2cm, y2cm)


def _merge_body(ks, kx1, ky1, kx2, ky2, scm, x1cm, y1cm, x2cm, y2cm,
                ob_hbm, os_hbm, ol_hbm,
                ks_v, kx1_v, ky1_v, kx2_v, ky2_v,
                pos_v, tmp16, sbuf, ob_v, os_v, ol_v):
    wid = lax.axis_index("s") * 2 + lax.axis_index("c")
    lane = lax.iota(jnp.int32, L)
    zero16 = jnp.zeros((L,), jnp.int32)

    @pl.when(wid == 0)
    def _():
        pltpu.sync_copy(ks, ks_v)
        pltpu.sync_copy(kx1, kx1_v)
        pltpu.sync_copy(ky1, ky1_v)
        pltpu.sync_copy(kx2, kx2_v)
        pltpu.sync_copy(ky2, ky2_v)

        def first_elem(hbm):
            pltpu.sync_copy(hbm.at[0, pl.ds(0, L)], tmp16)
            return plsc.load_gather(tmp16, [zero16])

        d_scr = first_elem(scm)
        d_x1 = first_elem(x1cm)
        d_y1 = first_elem(y1cm)
        d_x2 = first_elem(x2cm)
        d_y2 = first_elem(y2cm)

        def init_head(j, _):
            cls16 = j * L + lane
            hs = plsc.load_gather(ks_v, [cls16, zero16])
            sbuf[pl.ds(j * L, L)] = hs
            return 0

        lax.fori_loop(0, NC // L, init_head, 0)
        heads0 = tuple(sbuf[pl.ds(j * L, L)] for j in range(NC // L))

        for j in range(CAP // L):
            pos_v[pl.ds(j * L, L)] = jnp.zeros((L,), jnp.int32)

        def step(t, carry):
            (h0, h1, h2, h3, h4), last = carry
            lcls, lscr, lx1, ly1, lx2, ly2 = last
            m16 = jnp.maximum(jnp.maximum(jnp.maximum(h0, h1),
                                          jnp.maximum(h2, h3)), h4)
            msv = jnp.full((L,), jnp.max(m16))
            f0 = plsc.all_reduce_ffs(h0 == msv)
            f1 = plsc.all_reduce_ffs(h1 == msv)
            f2 = plsc.all_reduce_ffs(h2 == msv)
            f3 = plsc.all_reduce_ffs(h3 == msv)
            f4 = plsc.all_reduce_ffs(h4 == msv)
            cstar = jnp.where(
                f0 < L, f0,
                jnp.where(f1 < L, L + f1,
                          jnp.where(f2 < L, 2 * L + f2,
                                    jnp.where(f3 < L, 3 * L + f3, 4 * L + f4))))
            ex = msv <= -1e29
            csv = jnp.where(ex, 0, cstar)
            pv = plsc.load_gather(pos_v, [csv])
            bx1 = plsc.load_gather(kx1_v, [csv, pv])
            by1 = plsc.load_gather(ky1_v, [csv, pv])
            bx2 = plsc.load_gather(kx2_v, [csv, pv])
            by2 = plsc.load_gather(ky2_v, [csv, pv])
            pn = jnp.minimum(pv + 1, CAP - 1)
            nh_v = plsc.load_gather(ks_v, [csv, pn])
            plsc.store_scatter(pos_v, [csv], pn, mask=lane == 0)

            sel_lane = lane == (csv - (csv // L) * L)
            cchunk = csv // L
            h0 = jnp.where(sel_lane & (cchunk == 0), nh_v, h0)
            h1 = jnp.where(sel_lane & (cchunk == 1), nh_v, h1)
            h2 = jnp.where(sel_lane & (cchunk == 2), nh_v, h2)
            h3 = jnp.where(sel_lane & (cchunk == 3), nh_v, h3)
            h4 = jnp.where(sel_lane & (cchunk == 4), nh_v, h4)

            ncls = jnp.where(ex, lcls, csv + 1)
            nscr = jnp.where(ex, lscr, msv)
            nx1 = jnp.where(ex, lx1, bx1)
            ny1 = jnp.where(ex, ly1, by1)
            nx2 = jnp.where(ex, lx2, bx2)
            ny2 = jnp.where(ex, ly2, by2)

            tv = jnp.full((L,), t)
            plsc.store_scatter(os_v, [tv], nscr, mask=lane == 0)
            plsc.store_scatter(ol_v, [tv], ncls, mask=lane == 0)
            bval = jnp.where(lane == 0, nx1,
                             jnp.where(lane == 1, ny1,
                                       jnp.where(lane == 2, nx2, ny2)))
            plsc.store_scatter(ob_v, [t * 4 + lane], bval, mask=lane < 4)
            return ((h0, h1, h2, h3, h4),
                    (ncls, nscr, nx1, ny1, nx2, ny2))

        ones_i = jnp.full((L,), 1, jnp.int32)
        lax.fori_loop(0, DETS, step,
                      (heads0,
                       (ones_i, d_scr, d_x1, d_y1, d_x2, d_y2)))

        pltpu.sync_copy(ob_v, ob_hbm)
        pltpu.sync_copy(os_v, os_hbm)
        pltpu.sync_copy(ol_v, ol_hbm)


def _merge(ks, kx1, ky1, kx2, ky2, scm, x1cm, y1cm, x2cm, y2cm):
    mesh = plsc.VectorSubcoreMesh(core_axis_name="c", subcore_axis_name="s",
                                  num_cores=2, num_subcores=16)
    f32 = jnp.float32
    i32 = jnp.int32
    return pl.kernel(
        _merge_body,
        compiler_params=pltpu.CompilerParams(needs_layout_passes=False),
        out_type=[
            jax.ShapeDtypeStruct((CAP * 4,), f32),
            jax.ShapeDtypeStruct((CAP,), f32),
            jax.ShapeDtypeStruct((CAP,), i32),
        ],
        mesh=mesh,
        scratch_types=(
            [pltpu.VMEM((NC, CAP), f32)] * 5
            + [pltpu.VMEM((128,), i32),
               pltpu.VMEM((L,), f32),
               pltpu.VMEM((NC,), f32),
               pltpu.VMEM((CAP * 4,), f32), pltpu.VMEM((CAP,), f32),
               pltpu.VMEM((CAP,), i32)]
        ),
    )(ks, kx1, ky1, kx2, ky2, scm, x1cm, y1cm, x2cm, y2cm)


_DIAG = 0


def kernel(class_logits, box_regression, proposals):
    pad = NPAD - N
    lg = jnp.pad(class_logits, ((0, pad), (0, 0))).T
    brt = jnp.pad(box_regression.T, ((0, 0), (0, pad)))
    prop = jnp.pad(proposals.T, ((0, 4), (0, pad)))

    scm, x1cm, y1cm, x2cm, y2cm = _dense(lg, brt, prop)
    if _DIAG == 1:
        return (x1cm[0, :400].reshape(DETS, 4), scm[0, :DETS],
                y1cm[0, :DETS].astype(jnp.int32))
    ks, kx1, ky1, kx2, ky2 = _nms(scm, x1cm, y1cm, x2cm, y2cm)
    if _DIAG == 2:
        return (kx1[0, :400].reshape(DETS, 4), ks[0, :DETS],
                ky1[0, :DETS].astype(jnp.int32))
    ob, osc, olb = _merge(ks, kx1, ky1, kx2, ky2,
                          scm, x1cm, y1cm, x2cm, y2cm)
    det_boxes = ob[: DETS * 4].reshape(DETS, 4)
    return det_boxes, osc[:DETS], olb[:DETS]
